# baseline (speedup 1.0000x reference)
import functools

import jax
import jax.numpy as jnp
from jax import lax
from jax.experimental import pallas as pl
from jax.experimental.pallas import tpu as pltpu
from jax.experimental.pallas import tpu_sc as plsc

_BATCH = 16384
_FEAT = 512
_LANES = 16
_CHUNK = 64
_FVEC = _FEAT // _LANES


def _make_sc_partials():
    info = plsc.get_sparse_core_info()
    nc, ns = info.num_cores, info.num_subcores
    nw = nc * ns
    rows_per_w = _BATCH // nw
    nchunk = rows_per_w // _CHUNK

    mesh = plsc.VectorSubcoreMesh(core_axis_name="c", subcore_axis_name="s")

    @functools.partial(
        pl.kernel,
        mesh=mesh,
        compiler_params=pltpu.CompilerParams(needs_layout_passes=False),
        out_type=jax.ShapeDtypeStruct((nw, _LANES), jnp.float32),
        scratch_types=[
            pltpu.VMEM((_CHUNK,), jnp.int32),
            pltpu.VMEM((_CHUNK, _FEAT), jnp.float32),
            pltpu.VMEM((_CHUNK, _FEAT), jnp.float32),
            pltpu.VMEM((_LANES,), jnp.float32),
            pltpu.SemaphoreType.DMA,
        ],
    )
    def sc_kernel(x_hbm, lab_hbm, cen_hbm, out_hbm,
                  idx_v, x_v, c_v, part_v, sem):
        wid = lax.axis_index("s") * nc + lax.axis_index("c")
        base = wid * rows_per_w
        last_lane = lax.iota(jnp.int32, _LANES) == (_LANES - 1)

        def chunk_body(ci, total):
            rbase = base + ci * _CHUNK
            pltpu.sync_copy(lab_hbm.at[pl.ds(rbase, _CHUNK)], idx_v)
            gather = pltpu.async_copy(cen_hbm.at[idx_v], c_v, sem)
            pltpu.sync_copy(x_hbm.at[pl.ds(rbase, _CHUNK)], x_v)
            gather.wait()

            def row_body(row, tot):
                def feat_body(j, acc):
                    xv = x_v[row, pl.ds(j * _LANES, _LANES)]
                    cv = c_v[row, pl.ds(j * _LANES, _LANES)]
                    d = xv - cv
                    return acc + d * d

                acc = lax.fori_loop(
                    0, _FVEC, feat_body, jnp.zeros((_LANES,), jnp.float32))
                csum = plsc.cumsum(acc)
                csum = jnp.minimum(jnp.maximum(csum, 1e-12), 1e12)
                return tot + jnp.where(last_lane, csum, 0.0)

            return lax.fori_loop(0, _CHUNK, row_body, total)

        total = lax.fori_loop(
            0, nchunk, chunk_body, jnp.zeros((_LANES,), jnp.float32))
        part_v[...] = total
        pltpu.sync_copy(part_v, out_hbm.at[wid])

    return sc_kernel


_sc_partials = _make_sc_partials()


def _finish_body(p_ref, o_ref):
    o_ref[...] = jnp.sum(p_ref[...]).reshape(1, 1) * (1.0 / _BATCH)


def kernel(x, labels, centers):
    labels = labels.astype(jnp.int32)
    partials = _sc_partials(x, labels, centers)
    loss = pl.pallas_call(
        _finish_body,
        out_shape=jax.ShapeDtypeStruct((1, 1), jnp.float32),
    )(partials)
    return loss[0, 0]


# speedup vs baseline: 1.6104x; 1.6104x over previous
import functools

import jax
import jax.numpy as jnp
from jax import lax
from jax.experimental import pallas as pl
from jax.experimental.pallas import tpu as pltpu
from jax.experimental.pallas import tpu_sc as plsc

_BATCH = 16384
_FEAT = 512
_LANES = 16
_CHUNK = 64
_FVEC = _FEAT // _LANES


def _make_sc_partials():
    info = plsc.get_sparse_core_info()
    nc, ns = info.num_cores, info.num_subcores
    nw = nc * ns
    rows_per_w = _BATCH // nw
    nchunk = rows_per_w // _CHUNK

    mesh = plsc.VectorSubcoreMesh(core_axis_name="c", subcore_axis_name="s")

    @functools.partial(
        pl.kernel,
        mesh=mesh,
        compiler_params=pltpu.CompilerParams(needs_layout_passes=False),
        out_type=jax.ShapeDtypeStruct((nw, _LANES), jnp.float32),
        scratch_types=[
            pltpu.VMEM((_CHUNK,), jnp.int32),
            pltpu.VMEM((_CHUNK, _FEAT), jnp.float32),
            pltpu.VMEM((_CHUNK, _FEAT), jnp.float32),
            pltpu.VMEM((_LANES,), jnp.float32),
            pltpu.SemaphoreType.DMA,
        ],
    )
    def sc_kernel(x_hbm, lab_hbm, cen_hbm, out_hbm,
                  idx_v, x_v, c_v, part_v, sem):
        wid = lax.axis_index("s") * nc + lax.axis_index("c")
        base = wid * rows_per_w
        last_lane = lax.iota(jnp.int32, _LANES) == (_LANES - 1)

        def chunk_body(ci, total):
            rbase = base + ci * _CHUNK
            pltpu.sync_copy(lab_hbm.at[pl.ds(rbase, _CHUNK)], idx_v)
            gather = pltpu.async_copy(cen_hbm.at[idx_v], c_v, sem)
            pltpu.sync_copy(x_hbm.at[pl.ds(rbase, _CHUNK)], x_v)
            gather.wait()

            def row_body(row, tot):
                accs = [jnp.zeros((_LANES,), jnp.float32) for _ in range(4)]
                for j in range(_FVEC):
                    xv = x_v[row, pl.ds(j * _LANES, _LANES)]
                    cv = c_v[row, pl.ds(j * _LANES, _LANES)]
                    d = xv - cv
                    accs[j % 4] = accs[j % 4] + d * d
                acc = (accs[0] + accs[1]) + (accs[2] + accs[3])
                csum = plsc.cumsum(acc)
                csum = jnp.minimum(jnp.maximum(csum, 1e-12), 1e12)
                return tot + jnp.where(last_lane, csum, 0.0)

            return lax.fori_loop(0, _CHUNK, row_body, total)

        total = lax.fori_loop(
            0, nchunk, chunk_body, jnp.zeros((_LANES,), jnp.float32))
        part_v[...] = total
        pltpu.sync_copy(part_v, out_hbm.at[wid])

    return sc_kernel


_sc_partials = _make_sc_partials()


def _finish_body(p_ref, o_ref):
    o_ref[...] = jnp.sum(p_ref[...]).reshape(1, 1) * (1.0 / _BATCH)


def kernel(x, labels, centers):
    labels = labels.astype(jnp.int32)
    partials = _sc_partials(x, labels, centers)
    loss = pl.pallas_call(
        _finish_body,
        out_shape=jax.ShapeDtypeStruct((1, 1), jnp.float32),
    )(partials)
    return loss[0, 0]


# speedup vs baseline: 2.0595x; 1.2789x over previous
import functools

import jax
import jax.numpy as jnp
from jax import lax
from jax.experimental import pallas as pl
from jax.experimental.pallas import tpu as pltpu
from jax.experimental.pallas import tpu_sc as plsc

_BATCH = 16384
_FEAT = 512
_LANES = 16
_CHUNK = 32
_FVEC = _FEAT // _LANES


def _make_sc_partials():
    info = plsc.get_sparse_core_info()
    nc, ns = info.num_cores, info.num_subcores
    nw = nc * ns
    rows_per_w = _BATCH // nw
    nchunk = rows_per_w // _CHUNK

    mesh = plsc.VectorSubcoreMesh(core_axis_name="c", subcore_axis_name="s")

    @functools.partial(
        pl.kernel,
        mesh=mesh,
        compiler_params=pltpu.CompilerParams(needs_layout_passes=False),
        out_type=jax.ShapeDtypeStruct((nw, _LANES), jnp.float32),
        scratch_types=[
            pltpu.VMEM((nchunk, _CHUNK), jnp.int32),
            pltpu.VMEM((_CHUNK, _FEAT), jnp.float32),
            pltpu.VMEM((_CHUNK, _FEAT), jnp.float32),
            pltpu.VMEM((_CHUNK, _FEAT), jnp.float32),
            pltpu.VMEM((_CHUNK, _FEAT), jnp.float32),
            pltpu.VMEM((_LANES,), jnp.float32),
            pltpu.SemaphoreType.DMA,
            pltpu.SemaphoreType.DMA,
            pltpu.SemaphoreType.DMA,
            pltpu.SemaphoreType.DMA,
        ],
    )
    def sc_kernel(x_hbm, lab_hbm, cen_hbm, out_hbm,
                  idx_v, x0_v, x1_v, c0_v, c1_v, part_v,
                  semx0, semx1, semc0, semc1):
        wid = lax.axis_index("s") * nc + lax.axis_index("c")
        base = wid * rows_per_w
        last_lane = lax.iota(jnp.int32, _LANES) == (_LANES - 1)

        pltpu.sync_copy(lab_hbm.at[wid], idx_v)

        x_bufs, c_bufs = (x0_v, x1_v), (c0_v, c1_v)
        sem_x, sem_c = (semx0, semx1), (semc0, semc1)

        def issue(ci):
            b = ci % 2
            hx = pltpu.async_copy(
                x_hbm.at[pl.ds(base + ci * _CHUNK, _CHUNK)], x_bufs[b],
                sem_x[b])
            hc = pltpu.async_copy(
                cen_hbm.at[idx_v.at[ci]], c_bufs[b], sem_c[b])
            return hx, hc

        def row_body(x_v, c_v):
            def body(row, tot):
                accs = [jnp.zeros((_LANES,), jnp.float32) for _ in range(4)]
                for j in range(_FVEC):
                    xv = x_v[row, pl.ds(j * _LANES, _LANES)]
                    cv = c_v[row, pl.ds(j * _LANES, _LANES)]
                    d = xv - cv
                    accs[j % 4] = accs[j % 4] + d * d
                acc = (accs[0] + accs[1]) + (accs[2] + accs[3])
                csum = plsc.cumsum(acc)
                csum = jnp.minimum(jnp.maximum(csum, 1e-12), 1e12)
                return tot + jnp.where(last_lane, csum, 0.0)
            return body

        total = jnp.zeros((_LANES,), jnp.float32)
        pending = issue(0)
        for ci in range(nchunk):
            b = ci % 2
            hx, hc = pending
            if ci + 1 < nchunk:
                pending = issue(ci + 1)
            hx.wait()
            hc.wait()
            total = lax.fori_loop(
                0, _CHUNK, row_body(x_bufs[b], c_bufs[b]), total)

        part_v[...] = total
        pltpu.sync_copy(part_v, out_hbm.at[wid])

    return sc_kernel


_sc_partials = _make_sc_partials()


def _finish_body(p_ref, o_ref):
    o_ref[...] = jnp.sum(p_ref[...]).reshape(1, 1) * (1.0 / _BATCH)


def kernel(x, labels, centers):
    labels = labels.astype(jnp.int32).reshape(
        32, _BATCH // (32 * _CHUNK), _CHUNK)
    partials = _sc_partials(x, labels, centers)
    loss = pl.pallas_call(
        _finish_body,
        out_shape=jax.ShapeDtypeStruct((1, 1), jnp.float32),
    )(partials)
    return loss[0, 0]
